# Initial kernel scaffold; baseline (speedup 1.0000x reference)
import jax
import jax.numpy as jnp
from jax.experimental import pallas as pl


def kernel(x, edge_index, batch, params):
    raise NotImplementedError("write your pallas kernel here")


# speedup vs baseline: 8.8151x; 8.8151x over previous
import functools

import jax
import jax.numpy as jnp
from jax import lax
from jax.experimental import pallas as pl
from jax.experimental.pallas import tpu as pltpu
from jax.experimental.pallas import tpu_sc as plsc

N = 10000
E = 320000
G = 128
F_OUT = 64

NC = 2
NS = 16
NW = NC * NS

CHUNK = 128
EC = E // CHUNK
SLAB = (EC + NW - 1) // NW
SLAB = ((SLAB + 7) // 8) * 8
EC_PAD = SLAB * NW
N_PAD = ((N + NS * 8 - 1) // (NS * 8)) * (NS * 8)
RPT = N_PAD // NS

BR = 1000
GRID = N // BR


def _mm4_body(h_ref, wc_ref, wa_ref, wm1_ref, wm2_ref,
              ba_ref, bm1_ref, bm2_ref, bc_ref, u_ref, t_ref):
    h = h_ref[...]
    u_ref[...] = jnp.dot(h, wc_ref[...], preferred_element_type=jnp.float32)
    a = jnp.dot(h, wa_ref[...], preferred_element_type=jnp.float32) + ba_ref[...]
    m1 = jnp.dot(h, wm1_ref[...], preferred_element_type=jnp.float32) + bm1_ref[...]
    m2 = jnp.dot(h, wm2_ref[...], preferred_element_type=jnp.float32) + bm2_ref[...]
    t_ref[...] = a + m1 * m2 + bc_ref[...]


def _mm4(h, wc, wa, wm1, wm2, ba, bm1, bm2, bc):
    fin = h.shape[1]
    row = lambda i: (i, 0)
    full = lambda i: (0, 0)
    return pl.pallas_call(
        _mm4_body,
        grid=(GRID,),
        in_specs=[
            pl.BlockSpec((BR, fin), row),
            pl.BlockSpec((fin, F_OUT), full),
            pl.BlockSpec((fin, F_OUT), full),
            pl.BlockSpec((fin, F_OUT), full),
            pl.BlockSpec((fin, F_OUT), full),
            pl.BlockSpec((1, F_OUT), full),
            pl.BlockSpec((1, F_OUT), full),
            pl.BlockSpec((1, F_OUT), full),
            pl.BlockSpec((1, F_OUT), full),
        ],
        out_specs=[
            pl.BlockSpec((BR, F_OUT), row),
            pl.BlockSpec((BR, F_OUT), row),
        ],
        out_shape=[
            jax.ShapeDtypeStruct((N, F_OUT), jnp.float32),
            jax.ShapeDtypeStruct((N, F_OUT), jnp.float32),
        ],
    )(h, wc, wa, wm1, wm2, ba, bm1, bm2, bc)


def _sc_body(u_hbm, src_hbm, dst_hbm, zeros_hbm, out_hbm,
             sidx, didx, rows, acc, sem):
    c = lax.axis_index("c")
    s = lax.axis_index("s")
    wid = s * NC + c
    cnt = jnp.minimum(jnp.maximum(EC - SLAB * wid, 0), SLAB)

    pltpu.sync_copy(src_hbm.at[wid], sidx)
    pltpu.sync_copy(dst_hbm.at[wid], didx)
    pltpu.sync_copy(zeros_hbm, acc.at[pl.ds(s * RPT, RPT)])
    plsc.subcore_barrier()

    def step(j, carry):
        pltpu.async_copy(u_hbm.at[sidx.at[j]], rows, sem).wait()
        pltpu.sync_copy(rows, acc.at[didx.at[j]], add=True)
        return carry

    lax.fori_loop(0, cnt, step, jnp.int32(0))
    plsc.subcore_barrier()
    pltpu.sync_copy(acc.at[pl.ds(s * RPT, RPT)],
                    out_hbm.at[c, pl.ds(s * RPT, RPT)])


@functools.cache
def _sc_scatter():
    return functools.partial(
        pl.kernel,
        out_type=jax.ShapeDtypeStruct((NC, N_PAD, F_OUT), jnp.float32),
        mesh=plsc.VectorSubcoreMesh(core_axis_name="c", subcore_axis_name="s",
                                    num_cores=NC, num_subcores=NS),
        compiler_params=pltpu.CompilerParams(use_tc_tiling_on_sc=False),
        scratch_types=[
            pltpu.VMEM((SLAB, CHUNK), jnp.int32),
            pltpu.VMEM((SLAB, CHUNK), jnp.int32),
            pltpu.VMEM((CHUNK, F_OUT), jnp.float32),
            pltpu.VMEM_SHARED((N_PAD, F_OUT), jnp.float32),
            pltpu.SemaphoreType.DMA,
        ],
    )(_sc_body)


def _combine_body(t_ref, s0_ref, s1_ref, o_ref):
    o_ref[...] = jnp.maximum(t_ref[...] + s0_ref[0] + s1_ref[0], 0.0)


def _combine(t, s_part):
    row = lambda i: (i, 0)
    spec = pl.BlockSpec((BR, F_OUT), row)
    return pl.pallas_call(
        _combine_body,
        grid=(GRID,),
        in_specs=[
            spec,
            pl.BlockSpec((1, BR, F_OUT), lambda i: (0, i, 0)),
            pl.BlockSpec((1, BR, F_OUT), lambda i: (1, i, 0)),
        ],
        out_specs=spec,
        out_shape=jax.ShapeDtypeStruct((N, F_OUT), jnp.float32),
    )(t, s_part, s_part)


def _pool_body(h_ref, b_ref, w1_ref, b1_ref, w2_ref, b2_ref, o_ref,
               sums, cnts):
    i = pl.program_id(0)

    @pl.when(i == 0)
    def _init():
        sums[...] = jnp.zeros_like(sums)
        cnts[...] = jnp.zeros_like(cnts)

    bb = b_ref[0, 0, :]
    onehot = (bb[None, :] == lax.broadcasted_iota(jnp.int32, (G, BR), 0)
              ).astype(jnp.float32)
    sums[...] += jnp.dot(onehot, h_ref[...], preferred_element_type=jnp.float32)
    cnts[...] += jnp.sum(onehot, axis=1, keepdims=True)

    @pl.when(i == GRID - 1)
    def _final():
        pooled = sums[...] / jnp.maximum(cnts[...], 1.0)
        pooled = pooled * (1.0 / jnp.sqrt(1.0 + 1e-5))
        z = jnp.maximum(
            jnp.dot(pooled, w1_ref[...], preferred_element_type=jnp.float32)
            + b1_ref[...], 0.0)
        logits = (jnp.dot(z, w2_ref[...], preferred_element_type=jnp.float32)
                  + b2_ref[...])
        m = jnp.max(logits, axis=1, keepdims=True)
        e = jnp.exp(logits - m)
        lse = jnp.log(jnp.sum(e, axis=1, keepdims=True)) + m
        o_ref[...] = logits - lse


def _pool_fc(h, batch3d, w1, b1, w2, b2):
    row = lambda i: (i, 0)
    full = lambda i: (0, 0)
    return pl.pallas_call(
        _pool_body,
        grid=(GRID,),
        in_specs=[
            pl.BlockSpec((BR, F_OUT), row),
            pl.BlockSpec((1, 1, BR), lambda i: (i, 0, 0)),
            pl.BlockSpec((F_OUT, 32), full),
            pl.BlockSpec((1, 32), full),
            pl.BlockSpec((32, 10), full),
            pl.BlockSpec((1, 10), full),
        ],
        out_specs=pl.BlockSpec((G, 10), full),
        out_shape=jax.ShapeDtypeStruct((G, 10), jnp.float32),
        scratch_shapes=[
            pltpu.VMEM((G, F_OUT), jnp.float32),
            pltpu.VMEM((G, 1), jnp.float32),
        ],
    )(h, batch3d, w1, b1, w2, b2)


def kernel(x, edge_index, batch, params):
    p = params
    ei = edge_index.astype(jnp.int32)
    ei = jnp.pad(ei, ((0, 0), (0, EC_PAD * CHUNK - E)))
    src3 = ei[0].reshape(NW, SLAB, CHUNK)
    dst3 = ei[1].reshape(NW, SLAB, CHUNK)
    zeros = jnp.zeros((RPT, F_OUT), jnp.float32)
    batch3d = batch.astype(jnp.int32).reshape(GRID, 1, BR)

    r = lambda b: b.reshape(1, F_OUT)
    h = x
    for i in ("1", "2", "3"):
        u, t = _mm4(h,
                    p["Wc" + i], p["W" + i + "1"], p["W" + i + "2"],
                    p["W" + i + "3"],
                    r(p["b" + i + "1"]), r(p["b" + i + "2"]),
                    r(p["b" + i + "3"]), r(p["bc" + i]))
        s_part = _sc_scatter()(u, src3, dst3, zeros)
        h = _combine(t, s_part)

    return _pool_fc(h, batch3d, p["Wfc1"], p["bfc1"].reshape(1, 32),
                    p["Wfc2"], p["bfc2"].reshape(1, 10))


# speedup vs baseline: 12.7691x; 1.4486x over previous
import functools

import jax
import jax.numpy as jnp
from jax import lax
from jax.experimental import pallas as pl
from jax.experimental.pallas import tpu as pltpu
from jax.experimental.pallas import tpu_sc as plsc

N = 10000
E = 320000
G = 128
F_OUT = 64

NC = 2
NS = 16
NW = NC * NS

CHUNK = 128
EC = E // CHUNK
SLAB = (EC + NW - 1) // NW
SLAB = ((SLAB + 7) // 8) * 8
EC_PAD = SLAB * NW
N_PAD = ((N + NS * 8 - 1) // (NS * 8)) * (NS * 8)
RPT = N_PAD // NS

BR = 1000
GRID = N // BR


def _mm4_body(h_ref, wc_ref, wa_ref, wm1_ref, wm2_ref,
              ba_ref, bm1_ref, bm2_ref, bc_ref, u_ref, t_ref):
    h = h_ref[...]
    u_ref[...] = jnp.dot(h, wc_ref[...], preferred_element_type=jnp.float32)
    a = jnp.dot(h, wa_ref[...], preferred_element_type=jnp.float32) + ba_ref[...]
    m1 = jnp.dot(h, wm1_ref[...], preferred_element_type=jnp.float32) + bm1_ref[...]
    m2 = jnp.dot(h, wm2_ref[...], preferred_element_type=jnp.float32) + bm2_ref[...]
    t_ref[...] = a + m1 * m2 + bc_ref[...]


def _mm4(h, wc, wa, wm1, wm2, ba, bm1, bm2, bc):
    fin = h.shape[1]
    row = lambda i: (i, 0)
    full = lambda i: (0, 0)
    return pl.pallas_call(
        _mm4_body,
        grid=(GRID,),
        in_specs=[
            pl.BlockSpec((BR, fin), row),
            pl.BlockSpec((fin, F_OUT), full),
            pl.BlockSpec((fin, F_OUT), full),
            pl.BlockSpec((fin, F_OUT), full),
            pl.BlockSpec((fin, F_OUT), full),
            pl.BlockSpec((1, F_OUT), full),
            pl.BlockSpec((1, F_OUT), full),
            pl.BlockSpec((1, F_OUT), full),
            pl.BlockSpec((1, F_OUT), full),
        ],
        out_specs=[
            pl.BlockSpec((BR, F_OUT), row),
            pl.BlockSpec((BR, F_OUT), row),
        ],
        out_shape=[
            jax.ShapeDtypeStruct((N, F_OUT), jnp.float32),
            jax.ShapeDtypeStruct((N, F_OUT), jnp.float32),
        ],
    )(h, wc, wa, wm1, wm2, ba, bm1, bm2, bc)


K = 4
NG = SLAB // K


def _sc_body(u_hbm, src_hbm, dst_hbm, zeros_hbm, out_hbm,
             sidx, didx, rows, acc, gsem, ssem):
    c = lax.axis_index("c")
    s = lax.axis_index("s")
    wid = s * NC + c

    pltpu.sync_copy(src_hbm.at[wid], sidx)
    pltpu.sync_copy(dst_hbm.at[wid], didx)
    pltpu.sync_copy(zeros_hbm, acc.at[pl.ds(s * RPT, RPT)])
    plsc.subcore_barrier()

    def fire_gathers(g, buf):
        for b in range(K):
            pltpu.async_copy(u_hbm.at[sidx.at[g * K + b]],
                             rows.at[buf, pl.ds(b * CHUNK, CHUNK)], gsem)

    def fire_scatters(g, buf):
        for b in range(K):
            pltpu.async_copy(rows.at[buf, pl.ds(b * CHUNK, CHUNK)],
                             acc.at[didx.at[g * K + b]], ssem, add=True)

    def drain(sem, buf):
        pltpu.make_async_copy(u_hbm.at[pl.ds(0, K * CHUNK)],
                              rows.at[buf], sem).wait()

    fire_gathers(0, 0)

    @pl.loop(0, NG, step=2)
    def _grp(g):
        for p in range(2):
            gg = g + p
            cur, nxt = p, 1 - p
            drain(gsem, cur)

            @pl.when(gg > 0)
            def _():
                drain(ssem, nxt)

            @pl.when(gg + 1 < NG)
            def _():
                fire_gathers(gg + 1, nxt)

            fire_scatters(gg, cur)

    drain(ssem, 1)
    plsc.subcore_barrier()
    pltpu.sync_copy(acc.at[pl.ds(s * RPT, RPT)],
                    out_hbm.at[c, pl.ds(s * RPT, RPT)])


@functools.cache
def _sc_scatter():
    return functools.partial(
        pl.kernel,
        out_type=jax.ShapeDtypeStruct((NC, N_PAD, F_OUT), jnp.float32),
        mesh=plsc.VectorSubcoreMesh(core_axis_name="c", subcore_axis_name="s",
                                    num_cores=NC, num_subcores=NS),
        compiler_params=pltpu.CompilerParams(use_tc_tiling_on_sc=False),
        scratch_types=[
            pltpu.VMEM((SLAB, CHUNK), jnp.int32),
            pltpu.VMEM((SLAB, CHUNK), jnp.int32),
            pltpu.VMEM((2, K * CHUNK, F_OUT), jnp.float32),
            pltpu.VMEM_SHARED((N_PAD, F_OUT), jnp.float32),
            pltpu.SemaphoreType.DMA,
            pltpu.SemaphoreType.DMA,
        ],
    )(_sc_body)


def _combine_body(t_ref, s0_ref, s1_ref, o_ref):
    o_ref[...] = jnp.maximum(t_ref[...] + s0_ref[0] + s1_ref[0], 0.0)


def _combine(t, s_part):
    row = lambda i: (i, 0)
    spec = pl.BlockSpec((BR, F_OUT), row)
    return pl.pallas_call(
        _combine_body,
        grid=(GRID,),
        in_specs=[
            spec,
            pl.BlockSpec((1, BR, F_OUT), lambda i: (0, i, 0)),
            pl.BlockSpec((1, BR, F_OUT), lambda i: (1, i, 0)),
        ],
        out_specs=spec,
        out_shape=jax.ShapeDtypeStruct((N, F_OUT), jnp.float32),
    )(t, s_part, s_part)


def _pool_body(h_ref, b_ref, w1_ref, b1_ref, w2_ref, b2_ref, o_ref,
               sums, cnts):
    i = pl.program_id(0)

    @pl.when(i == 0)
    def _init():
        sums[...] = jnp.zeros_like(sums)
        cnts[...] = jnp.zeros_like(cnts)

    bb = b_ref[0, 0, :]
    onehot = (bb[None, :] == lax.broadcasted_iota(jnp.int32, (G, BR), 0)
              ).astype(jnp.float32)
    sums[...] += jnp.dot(onehot, h_ref[...], preferred_element_type=jnp.float32)
    cnts[...] += jnp.sum(onehot, axis=1, keepdims=True)

    @pl.when(i == GRID - 1)
    def _final():
        pooled = sums[...] / jnp.maximum(cnts[...], 1.0)
        pooled = pooled * (1.0 / jnp.sqrt(1.0 + 1e-5))
        z = jnp.maximum(
            jnp.dot(pooled, w1_ref[...], preferred_element_type=jnp.float32)
            + b1_ref[...], 0.0)
        logits = (jnp.dot(z, w2_ref[...], preferred_element_type=jnp.float32)
                  + b2_ref[...])
        m = jnp.max(logits, axis=1, keepdims=True)
        e = jnp.exp(logits - m)
        lse = jnp.log(jnp.sum(e, axis=1, keepdims=True)) + m
        o_ref[...] = logits - lse


def _pool_fc(h, batch3d, w1, b1, w2, b2):
    row = lambda i: (i, 0)
    full = lambda i: (0, 0)
    return pl.pallas_call(
        _pool_body,
        grid=(GRID,),
        in_specs=[
            pl.BlockSpec((BR, F_OUT), row),
            pl.BlockSpec((1, 1, BR), lambda i: (i, 0, 0)),
            pl.BlockSpec((F_OUT, 32), full),
            pl.BlockSpec((1, 32), full),
            pl.BlockSpec((32, 10), full),
            pl.BlockSpec((1, 10), full),
        ],
        out_specs=pl.BlockSpec((G, 10), full),
        out_shape=jax.ShapeDtypeStruct((G, 10), jnp.float32),
        scratch_shapes=[
            pltpu.VMEM((G, F_OUT), jnp.float32),
            pltpu.VMEM((G, 1), jnp.float32),
        ],
    )(h, batch3d, w1, b1, w2, b2)


def kernel(x, edge_index, batch, params):
    p = params
    ei = edge_index.astype(jnp.int32)
    pad_len = EC_PAD * CHUNK - E
    pad_iota = jnp.arange(pad_len, dtype=jnp.int32)
    src3 = jnp.concatenate([ei[0], pad_iota % N]).reshape(NW, SLAB, CHUNK)
    dst3 = jnp.concatenate([ei[1], N + pad_iota % (N_PAD - N)]
                           ).reshape(NW, SLAB, CHUNK)
    zeros = jnp.zeros((RPT, F_OUT), jnp.float32)
    batch3d = batch.astype(jnp.int32).reshape(GRID, 1, BR)

    r = lambda b: b.reshape(1, F_OUT)
    h = x
    for i in ("1", "2", "3"):
        u, t = _mm4(h,
                    p["Wc" + i], p["W" + i + "1"], p["W" + i + "2"],
                    p["W" + i + "3"],
                    r(p["b" + i + "1"]), r(p["b" + i + "2"]),
                    r(p["b" + i + "3"]), r(p["bc" + i]))
        s_part = _sc_scatter()(u, src3, dst3, zeros)
        h = _combine(t, s_part)

    return _pool_fc(h, batch3d, p["Wfc1"], p["bfc1"].reshape(1, 32),
                    p["Wfc2"], p["bfc2"].reshape(1, 10))


# speedup vs baseline: 14.0303x; 1.0988x over previous
import functools

import jax
import jax.numpy as jnp
from jax import lax
from jax.experimental import pallas as pl
from jax.experimental.pallas import tpu as pltpu
from jax.experimental.pallas import tpu_sc as plsc

N = 10000
E = 320000
G = 128
F_OUT = 64

NC = 2
NS = 16
NW = NC * NS

CHUNK = 128
EC = E // CHUNK
SLAB = (EC + NW - 1) // NW
SLAB = ((SLAB + 7) // 8) * 8
EC_PAD = SLAB * NW
N_PAD = ((N + NS * 8 - 1) // (NS * 8)) * (NS * 8)
RPT = N_PAD // NS

BR = 1000
GRID = N // BR


_ROW = lambda i: (i, 0)
_FULL = lambda i: (0, 0)


def _h_in_specs(fuse, fin):
    if fuse:
        return [
            pl.BlockSpec((BR, F_OUT), _ROW),
            pl.BlockSpec((1, BR, F_OUT), lambda i: (0, i, 0)),
            pl.BlockSpec((1, BR, F_OUT), lambda i: (1, i, 0)),
        ]
    return [pl.BlockSpec((BR, fin), _ROW)]


def _read_h(refs, fuse):
    if fuse:
        t_ref, s0_ref, s1_ref = refs
        return jnp.maximum(t_ref[...] + s0_ref[0] + s1_ref[0], 0.0)
    return refs[0][...]


def _mm_u(h_args, wc, fuse):
    fin = h_args[0].shape[-1]
    nh = len(h_args)

    def body(*refs):
        h = _read_h(refs[:nh], fuse)
        refs[-1][...] = jnp.dot(h, refs[nh][...],
                                preferred_element_type=jnp.float32)

    return pl.pallas_call(
        body,
        grid=(GRID,),
        in_specs=_h_in_specs(fuse, fin) + [pl.BlockSpec((fin, F_OUT), _FULL)],
        out_specs=pl.BlockSpec((BR, F_OUT), _ROW),
        out_shape=jax.ShapeDtypeStruct((N, F_OUT), jnp.float32),
    )(*h_args, wc)


def _mm_t(h_args, wa, wm1, wm2, ba, bm1, bm2, bc, fuse):
    fin = h_args[0].shape[-1]
    nh = len(h_args)

    def body(*refs):
        h = _read_h(refs[:nh], fuse)
        wa_r, wm1_r, wm2_r, ba_r, bm1_r, bm2_r, bc_r = refs[nh:nh + 7]
        a = jnp.dot(h, wa_r[...], preferred_element_type=jnp.float32) + ba_r[...]
        m1 = jnp.dot(h, wm1_r[...], preferred_element_type=jnp.float32) + bm1_r[...]
        m2 = jnp.dot(h, wm2_r[...], preferred_element_type=jnp.float32) + bm2_r[...]
        refs[-1][...] = a + m1 * m2 + bc_r[...]

    return pl.pallas_call(
        body,
        grid=(GRID,),
        in_specs=_h_in_specs(fuse, fin) + [
            pl.BlockSpec((fin, F_OUT), _FULL),
            pl.BlockSpec((fin, F_OUT), _FULL),
            pl.BlockSpec((fin, F_OUT), _FULL),
            pl.BlockSpec((1, F_OUT), _FULL),
            pl.BlockSpec((1, F_OUT), _FULL),
            pl.BlockSpec((1, F_OUT), _FULL),
            pl.BlockSpec((1, F_OUT), _FULL),
        ],
        out_specs=pl.BlockSpec((BR, F_OUT), _ROW),
        out_shape=jax.ShapeDtypeStruct((N, F_OUT), jnp.float32),
    )(*h_args, wa, wm1, wm2, ba, bm1, bm2, bc)


K = 4
NG = SLAB // K


def _sc_body(u_hbm, src_hbm, dst_hbm, zeros_hbm, out_hbm,
             sidx, didx, rows, acc, gsem, ssem):
    c = lax.axis_index("c")
    s = lax.axis_index("s")
    wid = s * NC + c

    pltpu.sync_copy(src_hbm.at[wid], sidx)
    pltpu.sync_copy(dst_hbm.at[wid], didx)
    pltpu.sync_copy(zeros_hbm, acc.at[pl.ds(s * RPT, RPT)])
    plsc.subcore_barrier()

    def fire_gathers(g, buf):
        for b in range(K):
            pltpu.async_copy(u_hbm.at[sidx.at[g * K + b]],
                             rows.at[buf, pl.ds(b * CHUNK, CHUNK)], gsem)

    def fire_scatters(g, buf):
        for b in range(K):
            pltpu.async_copy(rows.at[buf, pl.ds(b * CHUNK, CHUNK)],
                             acc.at[didx.at[g * K + b]], ssem, add=True)

    def drain(sem, buf):
        pltpu.make_async_copy(u_hbm.at[pl.ds(0, K * CHUNK)],
                              rows.at[buf], sem).wait()

    fire_gathers(0, 0)

    @pl.loop(0, NG, step=2)
    def _grp(g):
        for p in range(2):
            gg = g + p
            cur, nxt = p, 1 - p
            drain(gsem, cur)

            @pl.when(gg > 0)
            def _():
                drain(ssem, nxt)

            @pl.when(gg + 1 < NG)
            def _():
                fire_gathers(gg + 1, nxt)

            fire_scatters(gg, cur)

    drain(ssem, 1)
    plsc.subcore_barrier()
    pltpu.sync_copy(acc.at[pl.ds(s * RPT, RPT)],
                    out_hbm.at[c, pl.ds(s * RPT, RPT)])


@functools.cache
def _sc_scatter():
    return functools.partial(
        pl.kernel,
        out_type=jax.ShapeDtypeStruct((NC, N_PAD, F_OUT), jnp.float32),
        mesh=plsc.VectorSubcoreMesh(core_axis_name="c", subcore_axis_name="s",
                                    num_cores=NC, num_subcores=NS),
        compiler_params=pltpu.CompilerParams(use_tc_tiling_on_sc=False),
        scratch_types=[
            pltpu.VMEM((SLAB, CHUNK), jnp.int32),
            pltpu.VMEM((SLAB, CHUNK), jnp.int32),
            pltpu.VMEM((2, K * CHUNK, F_OUT), jnp.float32),
            pltpu.VMEM_SHARED((N_PAD, F_OUT), jnp.float32),
            pltpu.SemaphoreType.DMA,
            pltpu.SemaphoreType.DMA,
        ],
    )(_sc_body)


def _pool_body(t_ref, s0_ref, s1_ref, b_ref, w1_ref, b1_ref, w2_ref, b2_ref,
               o_ref, sums, cnts):
    i = pl.program_id(0)

    @pl.when(i == 0)
    def _init():
        sums[...] = jnp.zeros_like(sums)
        cnts[...] = jnp.zeros_like(cnts)

    h = jnp.maximum(t_ref[...] + s0_ref[0] + s1_ref[0], 0.0)
    bb = b_ref[0, 0, :]
    onehot = (bb[None, :] == lax.broadcasted_iota(jnp.int32, (G, BR), 0)
              ).astype(jnp.float32)
    sums[...] += jnp.dot(onehot, h, preferred_element_type=jnp.float32)
    cnts[...] += jnp.sum(onehot, axis=1, keepdims=True)

    @pl.when(i == GRID - 1)
    def _final():
        pooled = sums[...] / jnp.maximum(cnts[...], 1.0)
        pooled = pooled * (1.0 / jnp.sqrt(1.0 + 1e-5))
        z = jnp.maximum(
            jnp.dot(pooled, w1_ref[...], preferred_element_type=jnp.float32)
            + b1_ref[...], 0.0)
        logits = (jnp.dot(z, w2_ref[...], preferred_element_type=jnp.float32)
                  + b2_ref[...])
        m = jnp.max(logits, axis=1, keepdims=True)
        e = jnp.exp(logits - m)
        lse = jnp.log(jnp.sum(e, axis=1, keepdims=True)) + m
        o_ref[...] = logits - lse


def _pool_fc(t, s_part, batch3d, w1, b1, w2, b2):
    return pl.pallas_call(
        _pool_body,
        grid=(GRID,),
        in_specs=_h_in_specs(True, F_OUT) + [
            pl.BlockSpec((1, 1, BR), lambda i: (i, 0, 0)),
            pl.BlockSpec((F_OUT, 32), _FULL),
            pl.BlockSpec((1, 32), _FULL),
            pl.BlockSpec((32, 10), _FULL),
            pl.BlockSpec((1, 10), _FULL),
        ],
        out_specs=pl.BlockSpec((G, 10), _FULL),
        out_shape=jax.ShapeDtypeStruct((G, 10), jnp.float32),
        scratch_shapes=[
            pltpu.VMEM((G, F_OUT), jnp.float32),
            pltpu.VMEM((G, 1), jnp.float32),
        ],
    )(t, s_part, s_part, batch3d, w1, b1, w2, b2)


def kernel(x, edge_index, batch, params):
    p = params
    ei = edge_index.astype(jnp.int32)
    pad_len = EC_PAD * CHUNK - E
    pad_iota = jnp.arange(pad_len, dtype=jnp.int32)
    src3 = jnp.concatenate([ei[0], pad_iota % N]).reshape(NW, SLAB, CHUNK)
    dst3 = jnp.concatenate([ei[1], N + pad_iota % (N_PAD - N)]
                           ).reshape(NW, SLAB, CHUNK)
    zeros = jnp.zeros((RPT, F_OUT), jnp.float32)
    batch3d = batch.astype(jnp.int32).reshape(GRID, 1, BR)

    r = lambda b: b.reshape(1, F_OUT)
    h_args = (x,)
    fuse = False
    for i in ("1", "2", "3"):
        u = _mm_u(h_args, p["Wc" + i], fuse)
        s_part = _sc_scatter()(u, src3, dst3, zeros)
        t = _mm_t(h_args,
                  p["W" + i + "1"], p["W" + i + "2"], p["W" + i + "3"],
                  r(p["b" + i + "1"]), r(p["b" + i + "2"]),
                  r(p["b" + i + "3"]), r(p["bc" + i]), fuse)
        h_args = (t, s_part, s_part)
        fuse = True

    t, s_part, _ = h_args
    return _pool_fc(t, s_part, batch3d, p["Wfc1"], p["bfc1"].reshape(1, 32),
                    p["Wfc2"], p["bfc2"].reshape(1, 10))


# speedup vs baseline: 14.1587x; 1.0092x over previous
import functools

import jax
import jax.numpy as jnp
from jax import lax
from jax.experimental import pallas as pl
from jax.experimental.pallas import tpu as pltpu
from jax.experimental.pallas import tpu_sc as plsc

N = 10000
E = 320000
G = 128
F_OUT = 64

NC = 2
NS = 16
NW = NC * NS

CHUNK = 128
EC = E // CHUNK
SLAB = (EC + NW - 1) // NW
SLAB = ((SLAB + 7) // 8) * 8
EC_PAD = SLAB * NW
N_PAD = ((N + NS * 8 - 1) // (NS * 8)) * (NS * 8)
RPT = N_PAD // NS

BR = 1000
GRID = N // BR


_ROW = lambda i: (i, 0)
_FULL = lambda i: (0, 0)


def _h_in_specs(fuse, fin):
    if fuse:
        return [
            pl.BlockSpec((BR, F_OUT), _ROW),
            pl.BlockSpec((1, BR, F_OUT), lambda i: (0, i, 0)),
            pl.BlockSpec((1, BR, F_OUT), lambda i: (1, i, 0)),
        ]
    return [pl.BlockSpec((BR, fin), _ROW)]


def _read_h(refs, fuse):
    if fuse:
        t_ref, s0_ref, s1_ref = refs
        return jnp.maximum(t_ref[...] + s0_ref[0] + s1_ref[0], 0.0)
    return refs[0][...]


def _mm_u(h_args, wc, fuse):
    fin = h_args[0].shape[-1]
    nh = len(h_args)

    def body(*refs):
        h = _read_h(refs[:nh], fuse)
        refs[-1][...] = jnp.dot(h, refs[nh][...],
                                preferred_element_type=jnp.float32)

    return pl.pallas_call(
        body,
        grid=(GRID,),
        in_specs=_h_in_specs(fuse, fin) + [pl.BlockSpec((fin, F_OUT), _FULL)],
        out_specs=pl.BlockSpec((BR, F_OUT), _ROW),
        out_shape=jax.ShapeDtypeStruct((N, F_OUT), jnp.float32),
    )(*h_args, wc)


def _mm_t(h_args, wa, wm1, wm2, ba, bm1, bm2, bc, fuse):
    fin = h_args[0].shape[-1]
    nh = len(h_args)

    def body(*refs):
        h = _read_h(refs[:nh], fuse)
        wa_r, wm1_r, wm2_r, ba_r, bm1_r, bm2_r, bc_r = refs[nh:nh + 7]
        a = jnp.dot(h, wa_r[...], preferred_element_type=jnp.float32) + ba_r[...]
        m1 = jnp.dot(h, wm1_r[...], preferred_element_type=jnp.float32) + bm1_r[...]
        m2 = jnp.dot(h, wm2_r[...], preferred_element_type=jnp.float32) + bm2_r[...]
        refs[-1][...] = a + m1 * m2 + bc_r[...]

    return pl.pallas_call(
        body,
        grid=(GRID,),
        in_specs=_h_in_specs(fuse, fin) + [
            pl.BlockSpec((fin, F_OUT), _FULL),
            pl.BlockSpec((fin, F_OUT), _FULL),
            pl.BlockSpec((fin, F_OUT), _FULL),
            pl.BlockSpec((1, F_OUT), _FULL),
            pl.BlockSpec((1, F_OUT), _FULL),
            pl.BlockSpec((1, F_OUT), _FULL),
            pl.BlockSpec((1, F_OUT), _FULL),
        ],
        out_specs=pl.BlockSpec((BR, F_OUT), _ROW),
        out_shape=jax.ShapeDtypeStruct((N, F_OUT), jnp.float32),
    )(*h_args, wa, wm1, wm2, ba, bm1, bm2, bc)


K = 4
NG = SLAB // K


def _sc_body(u_hbm, edges_hbm, zeros_hbm, out_hbm,
             sidx, didx, rows, acc, gsem, ssem):
    c = lax.axis_index("c")
    s = lax.axis_index("s")
    wid = s * NC + c

    pltpu.sync_copy(edges_hbm.at[0, wid], sidx)
    pltpu.sync_copy(edges_hbm.at[1, wid], didx)
    pltpu.sync_copy(zeros_hbm, acc.at[pl.ds(s * RPT, RPT)])
    plsc.subcore_barrier()

    def fire_gathers(g, buf):
        for b in range(K):
            pltpu.async_copy(u_hbm.at[sidx.at[g * K + b]],
                             rows.at[buf, pl.ds(b * CHUNK, CHUNK)], gsem)

    def fire_scatters(g, buf):
        for b in range(K):
            pltpu.async_copy(rows.at[buf, pl.ds(b * CHUNK, CHUNK)],
                             acc.at[didx.at[g * K + b]], ssem, add=True)

    def drain(sem, buf):
        pltpu.make_async_copy(u_hbm.at[pl.ds(0, K * CHUNK)],
                              rows.at[buf], sem).wait()

    fire_gathers(0, 0)

    @pl.loop(0, NG, step=2)
    def _grp(g):
        for p in range(2):
            gg = g + p
            cur, nxt = p, 1 - p
            drain(gsem, cur)

            @pl.when(gg > 0)
            def _():
                drain(ssem, nxt)

            @pl.when(gg + 1 < NG)
            def _():
                fire_gathers(gg + 1, nxt)

            fire_scatters(gg, cur)

    drain(ssem, 1)
    plsc.subcore_barrier()
    pltpu.sync_copy(acc.at[pl.ds(s * RPT, RPT)],
                    out_hbm.at[c, pl.ds(s * RPT, RPT)])


@functools.cache
def _sc_scatter():
    return functools.partial(
        pl.kernel,
        out_type=jax.ShapeDtypeStruct((NC, N_PAD, F_OUT), jnp.float32),
        mesh=plsc.VectorSubcoreMesh(core_axis_name="c", subcore_axis_name="s",
                                    num_cores=NC, num_subcores=NS),
        compiler_params=pltpu.CompilerParams(use_tc_tiling_on_sc=False),
        scratch_types=[
            pltpu.VMEM((SLAB, CHUNK), jnp.int32),
            pltpu.VMEM((SLAB, CHUNK), jnp.int32),
            pltpu.VMEM((2, K * CHUNK, F_OUT), jnp.float32),
            pltpu.VMEM_SHARED((N_PAD, F_OUT), jnp.float32),
            pltpu.SemaphoreType.DMA,
            pltpu.SemaphoreType.DMA,
        ],
    )(_sc_body)


def _pool_body(t_ref, s0_ref, s1_ref, b_ref, w1_ref, b1_ref, w2_ref, b2_ref,
               o_ref, sums, cnts):
    i = pl.program_id(0)

    @pl.when(i == 0)
    def _init():
        sums[...] = jnp.zeros_like(sums)
        cnts[...] = jnp.zeros_like(cnts)

    h = jnp.maximum(t_ref[...] + s0_ref[0] + s1_ref[0], 0.0)
    bb = b_ref[0, 0, :]
    onehot = (bb[None, :] == lax.broadcasted_iota(jnp.int32, (G, BR), 0)
              ).astype(jnp.float32)
    sums[...] += jnp.dot(onehot, h, preferred_element_type=jnp.float32)
    cnts[...] += jnp.sum(onehot, axis=1, keepdims=True)

    @pl.when(i == GRID - 1)
    def _final():
        pooled = sums[...] / jnp.maximum(cnts[...], 1.0)
        pooled = pooled * (1.0 / jnp.sqrt(1.0 + 1e-5))
        z = jnp.maximum(
            jnp.dot(pooled, w1_ref[...], preferred_element_type=jnp.float32)
            + b1_ref[...], 0.0)
        logits = (jnp.dot(z, w2_ref[...], preferred_element_type=jnp.float32)
                  + b2_ref[...])
        m = jnp.max(logits, axis=1, keepdims=True)
        e = jnp.exp(logits - m)
        lse = jnp.log(jnp.sum(e, axis=1, keepdims=True)) + m
        o_ref[...] = logits - lse


def _pool_fc(t, s_part, batch3d, w1, b1, w2, b2):
    return pl.pallas_call(
        _pool_body,
        grid=(GRID,),
        in_specs=_h_in_specs(True, F_OUT) + [
            pl.BlockSpec((1, 1, BR), lambda i: (i, 0, 0)),
            pl.BlockSpec((F_OUT, 32), _FULL),
            pl.BlockSpec((1, 32), _FULL),
            pl.BlockSpec((32, 10), _FULL),
            pl.BlockSpec((1, 10), _FULL),
        ],
        out_specs=pl.BlockSpec((G, 10), _FULL),
        out_shape=jax.ShapeDtypeStruct((G, 10), jnp.float32),
        scratch_shapes=[
            pltpu.VMEM((G, F_OUT), jnp.float32),
            pltpu.VMEM((G, 1), jnp.float32),
        ],
    )(t, s_part, s_part, batch3d, w1, b1, w2, b2)


def kernel(x, edge_index, batch, params):
    p = params
    ei = edge_index.astype(jnp.int32)
    pad_len = EC_PAD * CHUNK - E
    pad_iota = jnp.arange(pad_len, dtype=jnp.int32)
    pads = jnp.stack([pad_iota % N, N + pad_iota % (N_PAD - N)])
    epad = jnp.concatenate([ei, pads], axis=1).reshape(2, NW, SLAB, CHUNK)
    zeros = jnp.zeros((RPT, F_OUT), jnp.float32)
    batch3d = batch.astype(jnp.int32).reshape(GRID, 1, BR)

    r = lambda b: b.reshape(1, F_OUT)
    h_args = (x,)
    fuse = False
    for i in ("1", "2", "3"):
        u = _mm_u(h_args, p["Wc" + i], fuse)
        s_part = _sc_scatter()(u, epad, zeros)
        t = _mm_t(h_args,
                  p["W" + i + "1"], p["W" + i + "2"], p["W" + i + "3"],
                  r(p["b" + i + "1"]), r(p["b" + i + "2"]),
                  r(p["b" + i + "3"]), r(p["bc" + i]), fuse)
        h_args = (t, s_part, s_part)
        fuse = True

    t, s_part, _ = h_args
    return _pool_fc(t, s_part, batch3d, p["Wfc1"], p["bfc1"].reshape(1, 32),
                    p["Wfc2"], p["bfc2"].reshape(1, 10))


# speedup vs baseline: 14.5580x; 1.0282x over previous
import functools

import jax
import jax.numpy as jnp
from jax import lax
from jax.experimental import pallas as pl
from jax.experimental.pallas import tpu as pltpu
from jax.experimental.pallas import tpu_sc as plsc

N = 10000
E = 320000
G = 128
F_OUT = 64

NC = 2
NS = 16
NW = NC * NS

CHUNK = 128
EC = E // CHUNK
SLAB = (EC + NW - 1) // NW
SLAB = ((SLAB + 7) // 8) * 8
EC_PAD = SLAB * NW
N_PAD = ((N + NS * 8 - 1) // (NS * 8)) * (NS * 8)
RPT = N_PAD // NS

BR = 1000
GRID = N // BR


_ROW = lambda i: (i, 0)
_FULL = lambda i: (0, 0)


def _h_in_specs(fuse, fin):
    if fuse:
        return [
            pl.BlockSpec((BR, F_OUT), _ROW),
            pl.BlockSpec((1, BR, F_OUT), lambda i: (0, i, 0)),
            pl.BlockSpec((1, BR, F_OUT), lambda i: (1, i, 0)),
        ]
    return [pl.BlockSpec((BR, fin), _ROW)]


def _read_h(refs, fuse):
    if fuse:
        t_ref, s0_ref, s1_ref = refs
        return jnp.maximum(t_ref[...] + s0_ref[0] + s1_ref[0], 0.0)
    return refs[0][...]


def _mm_u(h_args, wc, fuse):
    fin = h_args[0].shape[-1]
    nh = len(h_args)

    def body(*refs):
        h = _read_h(refs[:nh], fuse)
        refs[-1][...] = jnp.dot(h, refs[nh][...],
                                preferred_element_type=jnp.float32)

    return pl.pallas_call(
        body,
        grid=(GRID,),
        in_specs=_h_in_specs(fuse, fin) + [pl.BlockSpec((fin, F_OUT), _FULL)],
        out_specs=pl.BlockSpec((BR, F_OUT), _ROW),
        out_shape=jax.ShapeDtypeStruct((N, F_OUT), jnp.float32),
    )(*h_args, wc)


def _mm_t(h_args, wa, wm1, wm2, ba, bm1, bm2, bc, fuse):
    fin = h_args[0].shape[-1]
    nh = len(h_args)

    def body(*refs):
        h = _read_h(refs[:nh], fuse)
        wa_r, wm1_r, wm2_r, ba_r, bm1_r, bm2_r, bc_r = refs[nh:nh + 7]
        a = jnp.dot(h, wa_r[...], preferred_element_type=jnp.float32) + ba_r[...]
        m1 = jnp.dot(h, wm1_r[...], preferred_element_type=jnp.float32) + bm1_r[...]
        m2 = jnp.dot(h, wm2_r[...], preferred_element_type=jnp.float32) + bm2_r[...]
        refs[-1][...] = a + m1 * m2 + bc_r[...]

    return pl.pallas_call(
        body,
        grid=(GRID,),
        in_specs=_h_in_specs(fuse, fin) + [
            pl.BlockSpec((fin, F_OUT), _FULL),
            pl.BlockSpec((fin, F_OUT), _FULL),
            pl.BlockSpec((fin, F_OUT), _FULL),
            pl.BlockSpec((1, F_OUT), _FULL),
            pl.BlockSpec((1, F_OUT), _FULL),
            pl.BlockSpec((1, F_OUT), _FULL),
            pl.BlockSpec((1, F_OUT), _FULL),
        ],
        out_specs=pl.BlockSpec((BR, F_OUT), _ROW),
        out_shape=jax.ShapeDtypeStruct((N, F_OUT), jnp.float32),
    )(*h_args, wa, wm1, wm2, ba, bm1, bm2, bc)


K = 4
NG = SLAB // K


def _sc_body(u_hbm, edges_hbm, zeros_hbm, out_hbm,
             sidx, didx, rows, acc, gsem, ssem):
    c = lax.axis_index("c")
    s = lax.axis_index("s")
    wid = s * NC + c

    i0 = pltpu.async_copy(edges_hbm.at[0, wid], sidx, gsem)
    i1 = pltpu.async_copy(edges_hbm.at[1, wid], didx, gsem)
    z = pltpu.async_copy(zeros_hbm, acc.at[pl.ds(s * RPT, RPT)], ssem)
    i0.wait()
    i1.wait()

    def fire_gathers(g, buf):
        for b in range(K):
            pltpu.async_copy(u_hbm.at[sidx.at[g * K + b]],
                             rows.at[buf, pl.ds(b * CHUNK, CHUNK)], gsem)

    def fire_scatters(g, buf):
        for b in range(K):
            pltpu.async_copy(rows.at[buf, pl.ds(b * CHUNK, CHUNK)],
                             acc.at[didx.at[g * K + b]], ssem, add=True)

    def drain(sem, buf):
        pltpu.make_async_copy(u_hbm.at[pl.ds(0, K * CHUNK)],
                              rows.at[buf], sem).wait()

    fire_gathers(0, 0)
    z.wait()
    plsc.subcore_barrier()

    @pl.loop(0, NG, step=2)
    def _grp(g):
        for p in range(2):
            gg = g + p
            cur, nxt = p, 1 - p
            drain(gsem, cur)

            @pl.when(gg > 0)
            def _():
                drain(ssem, nxt)

            @pl.when(gg + 1 < NG)
            def _():
                fire_gathers(gg + 1, nxt)

            fire_scatters(gg, cur)

    drain(ssem, 1)
    plsc.subcore_barrier()
    pltpu.sync_copy(acc.at[pl.ds(s * RPT, RPT)],
                    out_hbm.at[c, pl.ds(s * RPT, RPT)])


@functools.cache
def _sc_scatter():
    return functools.partial(
        pl.kernel,
        out_type=jax.ShapeDtypeStruct((NC, N_PAD, F_OUT), jnp.float32),
        mesh=plsc.VectorSubcoreMesh(core_axis_name="c", subcore_axis_name="s",
                                    num_cores=NC, num_subcores=NS),
        compiler_params=pltpu.CompilerParams(use_tc_tiling_on_sc=False),
        scratch_types=[
            pltpu.VMEM((SLAB, CHUNK), jnp.int32),
            pltpu.VMEM((SLAB, CHUNK), jnp.int32),
            pltpu.VMEM((2, K * CHUNK, F_OUT), jnp.float32),
            pltpu.VMEM_SHARED((N_PAD, F_OUT), jnp.float32),
            pltpu.SemaphoreType.DMA,
            pltpu.SemaphoreType.DMA,
        ],
    )(_sc_body)


def _pool_body(t_ref, s0_ref, s1_ref, b_ref, w1_ref, b1_ref, w2_ref, b2_ref,
               o_ref, sums, cnts):
    i = pl.program_id(0)

    @pl.when(i == 0)
    def _init():
        sums[...] = jnp.zeros_like(sums)
        cnts[...] = jnp.zeros_like(cnts)

    h = jnp.maximum(t_ref[...] + s0_ref[0] + s1_ref[0], 0.0)
    bb = b_ref[0, 0, :]
    onehot = (bb[None, :] == lax.broadcasted_iota(jnp.int32, (G, BR), 0)
              ).astype(jnp.float32)
    sums[...] += jnp.dot(onehot, h, preferred_element_type=jnp.float32)
    cnts[...] += jnp.sum(onehot, axis=1, keepdims=True)

    @pl.when(i == GRID - 1)
    def _final():
        pooled = sums[...] / jnp.maximum(cnts[...], 1.0)
        pooled = pooled * (1.0 / jnp.sqrt(1.0 + 1e-5))
        z = jnp.maximum(
            jnp.dot(pooled, w1_ref[...], preferred_element_type=jnp.float32)
            + b1_ref[...], 0.0)
        logits = (jnp.dot(z, w2_ref[...], preferred_element_type=jnp.float32)
                  + b2_ref[...])
        m = jnp.max(logits, axis=1, keepdims=True)
        e = jnp.exp(logits - m)
        lse = jnp.log(jnp.sum(e, axis=1, keepdims=True)) + m
        o_ref[...] = logits - lse


def _pool_fc(t, s_part, batch3d, w1, b1, w2, b2):
    return pl.pallas_call(
        _pool_body,
        grid=(GRID,),
        in_specs=_h_in_specs(True, F_OUT) + [
            pl.BlockSpec((1, 1, BR), lambda i: (i, 0, 0)),
            pl.BlockSpec((F_OUT, 32), _FULL),
            pl.BlockSpec((1, 32), _FULL),
            pl.BlockSpec((32, 10), _FULL),
            pl.BlockSpec((1, 10), _FULL),
        ],
        out_specs=pl.BlockSpec((G, 10), _FULL),
        out_shape=jax.ShapeDtypeStruct((G, 10), jnp.float32),
        scratch_shapes=[
            pltpu.VMEM((G, F_OUT), jnp.float32),
            pltpu.VMEM((G, 1), jnp.float32),
        ],
    )(t, s_part, s_part, batch3d, w1, b1, w2, b2)


def kernel(x, edge_index, batch, params):
    p = params
    ei = edge_index.astype(jnp.int32)
    pad_len = EC_PAD * CHUNK - E
    pad_iota = jnp.arange(pad_len, dtype=jnp.int32)
    pads = jnp.stack([pad_iota % N, N + pad_iota % (N_PAD - N)])
    epad = jnp.concatenate([ei, pads], axis=1).reshape(2, NW, SLAB, CHUNK)
    zeros = jnp.zeros((RPT, F_OUT), jnp.float32)
    batch3d = batch.astype(jnp.int32).reshape(GRID, 1, BR)

    r = lambda b: b.reshape(1, F_OUT)
    h_args = (x,)
    fuse = False
    for i in ("1", "2", "3"):
        u = _mm_u(h_args, p["Wc" + i], fuse)
        s_part = _sc_scatter()(u, epad, zeros)
        t = _mm_t(h_args,
                  p["W" + i + "1"], p["W" + i + "2"], p["W" + i + "3"],
                  r(p["b" + i + "1"]), r(p["b" + i + "2"]),
                  r(p["b" + i + "3"]), r(p["bc" + i]), fuse)
        h_args = (t, s_part, s_part)
        fuse = True

    t, s_part, _ = h_args
    return _pool_fc(t, s_part, batch3d, p["Wfc1"], p["bfc1"].reshape(1, 32),
                    p["Wfc2"], p["bfc2"].reshape(1, 10))


# speedup vs baseline: 16.0342x; 1.1014x over previous
import functools

import jax
import jax.numpy as jnp
from jax import lax
from jax.experimental import pallas as pl
from jax.experimental.pallas import tpu as pltpu
from jax.experimental.pallas import tpu_sc as plsc

N = 10000
E = 320000
G = 128
F_OUT = 64

NC = 2
NS = 16
NW = NC * NS

CHUNK = 128
EC = E // CHUNK
SLAB = (EC + NW - 1) // NW
SLAB = ((SLAB + 7) // 8) * 8
EC_PAD = SLAB * NW
N_PAD = ((N + NS * 8 - 1) // (NS * 8)) * (NS * 8)
RPT = N_PAD // NS

BR = 1000
GRID = N // BR


_ROW = lambda i: (i, 0)
_FULL = lambda i: (0, 0)


def _h_in_specs(fuse, fin):
    if fuse:
        return [
            pl.BlockSpec((BR, F_OUT), _ROW),
            pl.BlockSpec((1, BR, 128), lambda i: (0, i, 0)),
            pl.BlockSpec((1, BR, 128), lambda i: (1, i, 0)),
        ]
    return [pl.BlockSpec((BR, fin), _ROW)]


def _read_h(refs, fuse):
    if fuse:
        t_ref, s0_ref, s1_ref = refs
        return jnp.maximum(
            t_ref[...] + s0_ref[0, :, :F_OUT] + s1_ref[0, :, :F_OUT], 0.0)
    return refs[0][...]


def _mm_u(h_args, wc, fuse):
    fin = h_args[0].shape[-1]
    nh = len(h_args)

    def body(*refs):
        h = _read_h(refs[:nh], fuse)
        refs[-1][...] = jnp.dot(h, refs[nh][...],
                                preferred_element_type=jnp.float32)

    return pl.pallas_call(
        body,
        grid=(GRID,),
        in_specs=_h_in_specs(fuse, fin) + [pl.BlockSpec((fin, F_OUT), _FULL)],
        out_specs=pl.BlockSpec((BR, F_OUT), _ROW),
        out_shape=jax.ShapeDtypeStruct((N, F_OUT), jnp.float32),
    )(*h_args, wc)


def _mm_t(h_args, wa, wm1, wm2, ba, bm1, bm2, bc, fuse):
    fin = h_args[0].shape[-1]
    nh = len(h_args)

    def body(*refs):
        h = _read_h(refs[:nh], fuse)
        wa_r, wm1_r, wm2_r, ba_r, bm1_r, bm2_r, bc_r = refs[nh:nh + 7]
        a = jnp.dot(h, wa_r[...], preferred_element_type=jnp.float32) + ba_r[...]
        m1 = jnp.dot(h, wm1_r[...], preferred_element_type=jnp.float32) + bm1_r[...]
        m2 = jnp.dot(h, wm2_r[...], preferred_element_type=jnp.float32) + bm2_r[...]
        refs[-1][...] = a + m1 * m2 + bc_r[...]

    return pl.pallas_call(
        body,
        grid=(GRID,),
        in_specs=_h_in_specs(fuse, fin) + [
            pl.BlockSpec((fin, F_OUT), _FULL),
            pl.BlockSpec((fin, F_OUT), _FULL),
            pl.BlockSpec((fin, F_OUT), _FULL),
            pl.BlockSpec((1, F_OUT), _FULL),
            pl.BlockSpec((1, F_OUT), _FULL),
            pl.BlockSpec((1, F_OUT), _FULL),
            pl.BlockSpec((1, F_OUT), _FULL),
        ],
        out_specs=pl.BlockSpec((BR, F_OUT), _ROW),
        out_shape=jax.ShapeDtypeStruct((N, F_OUT), jnp.float32),
    )(*h_args, wa, wm1, wm2, ba, bm1, bm2, bc)


K = 4
NG = SLAB // K


def _sc_body(u_hbm, edges_hbm, zeros_hbm, out_hbm,
             sidx, didx, rows, acc, gsem, ssem):
    c = lax.axis_index("c")
    s = lax.axis_index("s")
    wid = s * NC + c

    i0 = pltpu.async_copy(edges_hbm.at[0, wid], sidx, gsem)
    i1 = pltpu.async_copy(edges_hbm.at[1, wid], didx, gsem)
    z = pltpu.async_copy(zeros_hbm, acc.at[pl.ds(s * RPT, RPT)], ssem)
    i0.wait()
    i1.wait()

    def fire_gathers(g, buf):
        for b in range(K):
            pltpu.async_copy(u_hbm.at[sidx.at[g * K + b]],
                             rows.at[buf, pl.ds(b * CHUNK, CHUNK)], gsem)

    def fire_scatters(g, buf):
        for b in range(K):
            pltpu.async_copy(rows.at[buf, pl.ds(b * CHUNK, CHUNK)],
                             acc.at[didx.at[g * K + b]], ssem, add=True)

    def drain(sem, buf):
        pltpu.make_async_copy(u_hbm.at[pl.ds(0, K * CHUNK)],
                              rows.at[buf], sem).wait()

    fire_gathers(0, 0)
    z.wait()
    plsc.subcore_barrier()

    @pl.loop(0, NG, step=2)
    def _grp(g):
        for p in range(2):
            gg = g + p
            cur, nxt = p, 1 - p
            drain(gsem, cur)

            @pl.when(gg > 0)
            def _():
                drain(ssem, nxt)

            @pl.when(gg + 1 < NG)
            def _():
                fire_gathers(gg + 1, nxt)

            fire_scatters(gg, cur)

    drain(ssem, 1)
    plsc.subcore_barrier()
    pltpu.sync_copy(acc.at[pl.ds(s * RPT, RPT)],
                    out_hbm.at[c, pl.ds(s * RPT, RPT), pl.ds(0, F_OUT)])


@functools.cache
def _sc_scatter():
    return functools.partial(
        pl.kernel,
        out_type=jax.ShapeDtypeStruct((NC, N_PAD, 128), jnp.float32),
        mesh=plsc.VectorSubcoreMesh(core_axis_name="c", subcore_axis_name="s",
                                    num_cores=NC, num_subcores=NS),
        compiler_params=pltpu.CompilerParams(use_tc_tiling_on_sc=False),
        scratch_types=[
            pltpu.VMEM((SLAB, CHUNK), jnp.int32),
            pltpu.VMEM((SLAB, CHUNK), jnp.int32),
            pltpu.VMEM((2, K * CHUNK, F_OUT), jnp.float32),
            pltpu.VMEM_SHARED((N_PAD, F_OUT), jnp.float32),
            pltpu.SemaphoreType.DMA,
            pltpu.SemaphoreType.DMA,
        ],
    )(_sc_body)


def _pool_body(t_ref, s0_ref, s1_ref, b_ref, w1_ref, b1_ref, w2_ref, b2_ref,
               o_ref, sums, cnts):
    i = pl.program_id(0)

    @pl.when(i == 0)
    def _init():
        sums[...] = jnp.zeros_like(sums)
        cnts[...] = jnp.zeros_like(cnts)

    h = jnp.maximum(
        t_ref[...] + s0_ref[0, :, :F_OUT] + s1_ref[0, :, :F_OUT], 0.0)
    bb = b_ref[0, 0, :]
    onehot = (bb[None, :] == lax.broadcasted_iota(jnp.int32, (G, BR), 0)
              ).astype(jnp.float32)
    sums[...] += jnp.dot(onehot, h, preferred_element_type=jnp.float32)
    cnts[...] += jnp.sum(onehot, axis=1, keepdims=True)

    @pl.when(i == GRID - 1)
    def _final():
        pooled = sums[...] / jnp.maximum(cnts[...], 1.0)
        pooled = pooled * (1.0 / jnp.sqrt(1.0 + 1e-5))
        z = jnp.maximum(
            jnp.dot(pooled, w1_ref[...], preferred_element_type=jnp.float32)
            + b1_ref[...], 0.0)
        logits = (jnp.dot(z, w2_ref[...], preferred_element_type=jnp.float32)
                  + b2_ref[...])
        m = jnp.max(logits, axis=1, keepdims=True)
        e = jnp.exp(logits - m)
        lse = jnp.log(jnp.sum(e, axis=1, keepdims=True)) + m
        o_ref[...] = logits - lse


def _pool_fc(t, s_part, batch3d, w1, b1, w2, b2):
    return pl.pallas_call(
        _pool_body,
        grid=(GRID,),
        in_specs=_h_in_specs(True, F_OUT) + [
            pl.BlockSpec((1, 1, BR), lambda i: (i, 0, 0)),
            pl.BlockSpec((F_OUT, 32), _FULL),
            pl.BlockSpec((1, 32), _FULL),
            pl.BlockSpec((32, 10), _FULL),
            pl.BlockSpec((1, 10), _FULL),
        ],
        out_specs=pl.BlockSpec((G, 10), _FULL),
        out_shape=jax.ShapeDtypeStruct((G, 10), jnp.float32),
        scratch_shapes=[
            pltpu.VMEM((G, F_OUT), jnp.float32),
            pltpu.VMEM((G, 1), jnp.float32),
        ],
    )(t, s_part, s_part, batch3d, w1, b1, w2, b2)


def kernel(x, edge_index, batch, params):
    p = params
    ei = edge_index.astype(jnp.int32)
    pad_len = EC_PAD * CHUNK - E
    pad_iota = jnp.arange(pad_len, dtype=jnp.int32)
    pads = jnp.stack([pad_iota % N, N + pad_iota % (N_PAD - N)])
    epad = jnp.concatenate([ei, pads], axis=1).reshape(2, NW, SLAB, CHUNK)
    zeros = jnp.zeros((RPT, F_OUT), jnp.float32)
    batch3d = batch.astype(jnp.int32).reshape(GRID, 1, BR)

    r = lambda b: b.reshape(1, F_OUT)
    h_args = (x,)
    fuse = False
    for i in ("1", "2", "3"):
        u = _mm_u(h_args, p["Wc" + i], fuse)
        s_part = _sc_scatter()(u, epad, zeros)
        t = _mm_t(h_args,
                  p["W" + i + "1"], p["W" + i + "2"], p["W" + i + "3"],
                  r(p["b" + i + "1"]), r(p["b" + i + "2"]),
                  r(p["b" + i + "3"]), r(p["bc" + i]), fuse)
        h_args = (t, s_part, s_part)
        fuse = True

    t, s_part, _ = h_args
    return _pool_fc(t, s_part, batch3d, p["Wfc1"], p["bfc1"].reshape(1, 32),
                    p["Wfc2"], p["bfc2"].reshape(1, 10))


# speedup vs baseline: 16.4398x; 1.0253x over previous
import functools

import jax
import jax.numpy as jnp
from jax import lax
from jax.experimental import pallas as pl
from jax.experimental.pallas import tpu as pltpu
from jax.experimental.pallas import tpu_sc as plsc

N = 10000
E = 320000
G = 128
F_OUT = 64

NC = 2
NS = 16
NW = NC * NS

CHUNK = 128
EC = E // CHUNK
SLAB = (EC + NW - 1) // NW
SLAB = ((SLAB + 7) // 8) * 8
EC_PAD = SLAB * NW
N_PAD = ((N + NS * 8 - 1) // (NS * 8)) * (NS * 8)
RPT = N_PAD // NS

BR = 1000
GRID = N // BR


_ROW = lambda i: (i, 0)
_FULL = lambda i: (0, 0)


def _h_in_specs(fuse, fin):
    if fuse:
        return [
            pl.BlockSpec((BR, F_OUT), _ROW),
            pl.BlockSpec((1, BR, 128), lambda i: (0, i, 0)),
            pl.BlockSpec((1, BR, 128), lambda i: (1, i, 0)),
        ]
    return [pl.BlockSpec((BR, fin), _ROW)]


def _read_h(refs, fuse):
    if fuse:
        t_ref, s0_ref, s1_ref = refs
        return jnp.maximum(
            t_ref[...] + s0_ref[0, :, :F_OUT] + s1_ref[0, :, :F_OUT], 0.0)
    return refs[0][...]


def _mm_u(h_args, wc, fuse):
    fin = h_args[0].shape[-1]
    nh = len(h_args)

    def body(*refs):
        h = _read_h(refs[:nh], fuse)
        refs[-1][...] = jnp.dot(h, refs[nh][...],
                                preferred_element_type=jnp.float32)

    return pl.pallas_call(
        body,
        grid=(GRID,),
        in_specs=_h_in_specs(fuse, fin) + [pl.BlockSpec((fin, 128), _FULL)],
        out_specs=pl.BlockSpec((BR, 128), _ROW),
        out_shape=jax.ShapeDtypeStruct((N, 128), jnp.float32),
    )(*h_args, wc)


def _mm_t(h_args, wa, wm1, wm2, ba, bm1, bm2, bc, fuse):
    fin = h_args[0].shape[-1]
    nh = len(h_args)

    def body(*refs):
        h = _read_h(refs[:nh], fuse)
        wa_r, wm1_r, wm2_r, ba_r, bm1_r, bm2_r, bc_r = refs[nh:nh + 7]
        a = jnp.dot(h, wa_r[...], preferred_element_type=jnp.float32) + ba_r[...]
        m1 = jnp.dot(h, wm1_r[...], preferred_element_type=jnp.float32) + bm1_r[...]
        m2 = jnp.dot(h, wm2_r[...], preferred_element_type=jnp.float32) + bm2_r[...]
        refs[-1][...] = a + m1 * m2 + bc_r[...]

    return pl.pallas_call(
        body,
        grid=(GRID,),
        in_specs=_h_in_specs(fuse, fin) + [
            pl.BlockSpec((fin, F_OUT), _FULL),
            pl.BlockSpec((fin, F_OUT), _FULL),
            pl.BlockSpec((fin, F_OUT), _FULL),
            pl.BlockSpec((1, F_OUT), _FULL),
            pl.BlockSpec((1, F_OUT), _FULL),
            pl.BlockSpec((1, F_OUT), _FULL),
            pl.BlockSpec((1, F_OUT), _FULL),
        ],
        out_specs=pl.BlockSpec((BR, F_OUT), _ROW),
        out_shape=jax.ShapeDtypeStruct((N, F_OUT), jnp.float32),
    )(*h_args, wa, wm1, wm2, ba, bm1, bm2, bc)


K = 4
NG = SLAB // K


def _sc_body(u_hbm, edges_hbm, zeros_hbm, out_hbm,
             sidx, didx, rows, acc, gsem, ssem):
    c = lax.axis_index("c")
    s = lax.axis_index("s")
    wid = s * NC + c

    i0 = pltpu.async_copy(edges_hbm.at[0, wid], sidx, gsem)
    i1 = pltpu.async_copy(edges_hbm.at[1, wid], didx, gsem)
    z = pltpu.async_copy(zeros_hbm, acc.at[pl.ds(s * RPT, RPT)], ssem)
    i0.wait()
    i1.wait()

    def fire_gathers(g, buf):
        for b in range(K):
            pltpu.async_copy(u_hbm.at[sidx.at[g * K + b]],
                             rows.at[buf, pl.ds(b * CHUNK, CHUNK)], gsem)

    def fire_scatters(g, buf):
        for b in range(K):
            pltpu.async_copy(rows.at[buf, pl.ds(b * CHUNK, CHUNK)],
                             acc.at[didx.at[g * K + b]], ssem, add=True)

    def drain(sem, buf):
        pltpu.make_async_copy(u_hbm.at[pl.ds(0, K * CHUNK)],
                              rows.at[buf], sem).wait()

    fire_gathers(0, 0)
    z.wait()
    plsc.subcore_barrier()

    @pl.loop(0, NG, step=2)
    def _grp(g):
        for p in range(2):
            gg = g + p
            cur, nxt = p, 1 - p
            drain(gsem, cur)

            @pl.when(gg > 0)
            def _():
                drain(ssem, nxt)

            @pl.when(gg + 1 < NG)
            def _():
                fire_gathers(gg + 1, nxt)

            fire_scatters(gg, cur)

    drain(ssem, 1)
    plsc.subcore_barrier()
    pltpu.sync_copy(acc.at[pl.ds(s * RPT, RPT)],
                    out_hbm.at[c, pl.ds(s * RPT, RPT), pl.ds(0, F_OUT)])


@functools.cache
def _sc_scatter():
    return functools.partial(
        pl.kernel,
        out_type=jax.ShapeDtypeStruct((NC, N_PAD, 128), jnp.float32),
        mesh=plsc.VectorSubcoreMesh(core_axis_name="c", subcore_axis_name="s",
                                    num_cores=NC, num_subcores=NS),
        compiler_params=pltpu.CompilerParams(use_tc_tiling_on_sc=False),
        scratch_types=[
            pltpu.VMEM((SLAB, CHUNK), jnp.int32),
            pltpu.VMEM((SLAB, CHUNK), jnp.int32),
            pltpu.VMEM((2, K * CHUNK, F_OUT), jnp.float32),
            pltpu.VMEM_SHARED((N_PAD, F_OUT), jnp.float32),
            pltpu.SemaphoreType.DMA,
            pltpu.SemaphoreType.DMA,
        ],
    )(_sc_body)


def _pool_body(t_ref, s0_ref, s1_ref, b_ref, w1_ref, b1_ref, w2_ref, b2_ref,
               o_ref, sums, cnts):
    i = pl.program_id(0)

    @pl.when(i == 0)
    def _init():
        sums[...] = jnp.zeros_like(sums)
        cnts[...] = jnp.zeros_like(cnts)

    h = jnp.maximum(
        t_ref[...] + s0_ref[0, :, :F_OUT] + s1_ref[0, :, :F_OUT], 0.0)
    bb = b_ref[0, 0, :]
    onehot = (bb[None, :] == lax.broadcasted_iota(jnp.int32, (G, BR), 0)
              ).astype(jnp.float32)
    sums[...] += jnp.dot(onehot, h, preferred_element_type=jnp.float32)
    cnts[...] += jnp.sum(onehot, axis=1, keepdims=True)

    @pl.when(i == GRID - 1)
    def _final():
        pooled = sums[...] / jnp.maximum(cnts[...], 1.0)
        pooled = pooled * (1.0 / jnp.sqrt(1.0 + 1e-5))
        z = jnp.maximum(
            jnp.dot(pooled, w1_ref[...], preferred_element_type=jnp.float32)
            + b1_ref[...], 0.0)
        logits = (jnp.dot(z, w2_ref[...], preferred_element_type=jnp.float32)
                  + b2_ref[...])
        m = jnp.max(logits, axis=1, keepdims=True)
        e = jnp.exp(logits - m)
        lse = jnp.log(jnp.sum(e, axis=1, keepdims=True)) + m
        o_ref[...] = logits - lse


def _pool_fc(t, s_part, batch3d, w1, b1, w2, b2):
    return pl.pallas_call(
        _pool_body,
        grid=(GRID,),
        in_specs=_h_in_specs(True, F_OUT) + [
            pl.BlockSpec((1, 1, BR), lambda i: (i, 0, 0)),
            pl.BlockSpec((F_OUT, 32), _FULL),
            pl.BlockSpec((1, 32), _FULL),
            pl.BlockSpec((32, 10), _FULL),
            pl.BlockSpec((1, 10), _FULL),
        ],
        out_specs=pl.BlockSpec((G, 10), _FULL),
        out_shape=jax.ShapeDtypeStruct((G, 10), jnp.float32),
        scratch_shapes=[
            pltpu.VMEM((G, F_OUT), jnp.float32),
            pltpu.VMEM((G, 1), jnp.float32),
        ],
    )(t, s_part, s_part, batch3d, w1, b1, w2, b2)


def kernel(x, edge_index, batch, params):
    p = params
    ei = edge_index.astype(jnp.int32)
    pad_len = EC_PAD * CHUNK - E
    pad_iota = jnp.arange(pad_len, dtype=jnp.int32)
    srcs = jnp.concatenate([ei[0], pad_iota % N]) * 2
    dsts = jnp.concatenate([ei[1], N + pad_iota % (N_PAD - N)])
    epad = jnp.stack([srcs, dsts]).reshape(2, NW, SLAB, CHUNK)
    zeros = jnp.zeros((RPT, F_OUT), jnp.float32)
    batch3d = batch.astype(jnp.int32).reshape(GRID, 1, BR)

    r = lambda b: b.reshape(1, F_OUT)
    h_args = (x,)
    fuse = False
    for i in ("1", "2", "3"):
        wc_pad = jnp.pad(p["Wc" + i], ((0, 0), (0, 128 - F_OUT)))
        u = _mm_u(h_args, wc_pad, fuse).reshape(2 * N, F_OUT)
        s_part = _sc_scatter()(u, epad, zeros)
        t = _mm_t(h_args,
                  p["W" + i + "1"], p["W" + i + "2"], p["W" + i + "3"],
                  r(p["b" + i + "1"]), r(p["b" + i + "2"]),
                  r(p["b" + i + "3"]), r(p["bc" + i]), fuse)
        h_args = (t, s_part, s_part)
        fuse = True

    t, s_part, _ = h_args
    return _pool_fc(t, s_part, batch3d, p["Wfc1"], p["bfc1"].reshape(1, 32),
                    p["Wfc2"], p["bfc2"].reshape(1, 10))


# speedup vs baseline: 17.3611x; 1.0560x over previous
import functools

import jax
import jax.numpy as jnp
from jax import lax
from jax.experimental import pallas as pl
from jax.experimental.pallas import tpu as pltpu
from jax.experimental.pallas import tpu_sc as plsc

N = 10000
E = 320000
G = 128
F_OUT = 64

NC = 2
NS = 16
NW = NC * NS

CHUNK = 128
EC = E // CHUNK
SLAB = (EC + NW - 1) // NW
SLAB = ((SLAB + 7) // 8) * 8
EC_PAD = SLAB * NW
N_PAD = ((N + NS * 8 - 1) // (NS * 8)) * (NS * 8)
RPT = N_PAD // NS

BR = 1000
GRID = N // BR


_ROW = lambda i: (i, 0)
_FULL = lambda i: (0, 0)


def _h_in_specs(fuse, fin):
    if fuse:
        return [
            pl.BlockSpec((BR, F_OUT), _ROW),
            pl.BlockSpec((1, BR, 128), lambda i: (0, i, 0)),
            pl.BlockSpec((1, BR, 128), lambda i: (1, i, 0)),
        ]
    return [pl.BlockSpec((BR, fin), _ROW)]


def _read_h(refs, fuse):
    if fuse:
        t_ref, s0_ref, s1_ref = refs
        return jnp.maximum(
            t_ref[...] + s0_ref[0, :, :F_OUT] + s1_ref[0, :, :F_OUT], 0.0)
    return refs[0][...]


def _mm_u(h_args, wc, fuse):
    fin = h_args[0].shape[-1]
    nh = len(h_args)

    def body(*refs):
        h = _read_h(refs[:nh], fuse)
        refs[-1][...] = jnp.dot(h, refs[nh][...],
                                preferred_element_type=jnp.float32)

    if fuse:
        h_specs = [
            pl.BlockSpec((N, F_OUT), _ROW),
            pl.BlockSpec((1, N, 128), lambda i: (0, 0, 0)),
            pl.BlockSpec((1, N, 128), lambda i: (1, 0, 0)),
        ]
    else:
        h_specs = [pl.BlockSpec((N, fin), _ROW)]
    return pl.pallas_call(
        body,
        grid=(1,),
        in_specs=h_specs + [pl.BlockSpec((fin, 128), _FULL)],
        out_specs=pl.BlockSpec((N, 128), _ROW),
        out_shape=jax.ShapeDtypeStruct((N, 128), jnp.float32),
    )(*h_args, wc)


def _mm_t(h_args, wa, wm1, wm2, ba, bm1, bm2, bc, fuse):
    fin = h_args[0].shape[-1]
    nh = len(h_args)

    def body(*refs):
        h = _read_h(refs[:nh], fuse)
        wa_r, wm1_r, wm2_r, ba_r, bm1_r, bm2_r, bc_r = refs[nh:nh + 7]
        a = jnp.dot(h, wa_r[...], preferred_element_type=jnp.float32) + ba_r[...]
        m1 = jnp.dot(h, wm1_r[...], preferred_element_type=jnp.float32) + bm1_r[...]
        m2 = jnp.dot(h, wm2_r[...], preferred_element_type=jnp.float32) + bm2_r[...]
        refs[-1][...] = a + m1 * m2 + bc_r[...]

    return pl.pallas_call(
        body,
        grid=(GRID,),
        in_specs=_h_in_specs(fuse, fin) + [
            pl.BlockSpec((fin, F_OUT), _FULL),
            pl.BlockSpec((fin, F_OUT), _FULL),
            pl.BlockSpec((fin, F_OUT), _FULL),
            pl.BlockSpec((1, F_OUT), _FULL),
            pl.BlockSpec((1, F_OUT), _FULL),
            pl.BlockSpec((1, F_OUT), _FULL),
            pl.BlockSpec((1, F_OUT), _FULL),
        ],
        out_specs=pl.BlockSpec((BR, F_OUT), _ROW),
        out_shape=jax.ShapeDtypeStruct((N, F_OUT), jnp.float32),
    )(*h_args, wa, wm1, wm2, ba, bm1, bm2, bc)


K = 4
NG = SLAB // K


def _sc_body(u_hbm, edges_hbm, zeros_hbm, out_hbm,
             sidx, didx, rows, acc, gsem, ssem):
    c = lax.axis_index("c")
    s = lax.axis_index("s")
    wid = s * NC + c

    i0 = pltpu.async_copy(edges_hbm.at[0, wid], sidx, gsem)
    i1 = pltpu.async_copy(edges_hbm.at[1, wid], didx, gsem)
    z = pltpu.async_copy(zeros_hbm, acc.at[pl.ds(s * RPT, RPT)], ssem)
    i0.wait()
    i1.wait()

    def fire_gathers(g, buf):
        for b in range(K):
            pltpu.async_copy(u_hbm.at[sidx.at[g * K + b]],
                             rows.at[buf, pl.ds(b * CHUNK, CHUNK)], gsem)

    def fire_scatters(g, buf):
        for b in range(K):
            pltpu.async_copy(rows.at[buf, pl.ds(b * CHUNK, CHUNK)],
                             acc.at[didx.at[g * K + b]], ssem, add=True)

    def drain(sem, buf):
        pltpu.make_async_copy(u_hbm.at[pl.ds(0, K * CHUNK)],
                              rows.at[buf], sem).wait()

    fire_gathers(0, 0)
    z.wait()
    plsc.subcore_barrier()

    @pl.loop(0, NG, step=2)
    def _grp(g):
        for p in range(2):
            gg = g + p
            cur, nxt = p, 1 - p
            drain(gsem, cur)

            @pl.when(gg > 0)
            def _():
                drain(ssem, nxt)

            @pl.when(gg + 1 < NG)
            def _():
                fire_gathers(gg + 1, nxt)

            fire_scatters(gg, cur)

    drain(ssem, 1)
    plsc.subcore_barrier()
    pltpu.sync_copy(acc.at[pl.ds(s * RPT, RPT)],
                    out_hbm.at[c, pl.ds(s * RPT, RPT), pl.ds(0, F_OUT)])


@functools.cache
def _sc_scatter():
    return functools.partial(
        pl.kernel,
        out_type=jax.ShapeDtypeStruct((NC, N_PAD, 128), jnp.float32),
        mesh=plsc.VectorSubcoreMesh(core_axis_name="c", subcore_axis_name="s",
                                    num_cores=NC, num_subcores=NS),
        compiler_params=pltpu.CompilerParams(use_tc_tiling_on_sc=False),
        scratch_types=[
            pltpu.VMEM((SLAB, CHUNK), jnp.int32),
            pltpu.VMEM((SLAB, CHUNK), jnp.int32),
            pltpu.VMEM((2, K * CHUNK, F_OUT), jnp.float32),
            pltpu.VMEM_SHARED((N_PAD, F_OUT), jnp.float32),
            pltpu.SemaphoreType.DMA,
            pltpu.SemaphoreType.DMA,
        ],
    )(_sc_body)


def _pool_body(t_ref, s0_ref, s1_ref, b_ref, w1_ref, b1_ref, w2_ref, b2_ref,
               o_ref):
    h = jnp.maximum(
        t_ref[...] + s0_ref[0, :, :F_OUT] + s1_ref[0, :, :F_OUT], 0.0)
    bb = b_ref[0]
    onehot = (bb[None, :] == lax.broadcasted_iota(jnp.int32, (G, N), 0)
              ).astype(jnp.float32)
    sums = jnp.dot(onehot, h, preferred_element_type=jnp.float32)
    cnts = jnp.sum(onehot, axis=1, keepdims=True)
    pooled = sums / jnp.maximum(cnts, 1.0)
    pooled = pooled * (1.0 / jnp.sqrt(1.0 + 1e-5))
    z = jnp.maximum(
        jnp.dot(pooled, w1_ref[...], preferred_element_type=jnp.float32)
        + b1_ref[...], 0.0)
    logits = (jnp.dot(z, w2_ref[...], preferred_element_type=jnp.float32)
              + b2_ref[...])
    m = jnp.max(logits, axis=1, keepdims=True)
    e = jnp.exp(logits - m)
    lse = jnp.log(jnp.sum(e, axis=1, keepdims=True)) + m
    o_ref[...] = logits - lse


def _pool_fc(t, s_part, batch2d, w1, b1, w2, b2):
    return pl.pallas_call(
        _pool_body,
        grid=(1,),
        in_specs=[
            pl.BlockSpec((N, F_OUT), _ROW),
            pl.BlockSpec((1, N, 128), lambda i: (0, 0, 0)),
            pl.BlockSpec((1, N, 128), lambda i: (1, 0, 0)),
            pl.BlockSpec((1, N), _FULL),
            pl.BlockSpec((F_OUT, 32), _FULL),
            pl.BlockSpec((1, 32), _FULL),
            pl.BlockSpec((32, 10), _FULL),
            pl.BlockSpec((1, 10), _FULL),
        ],
        out_specs=pl.BlockSpec((G, 10), _FULL),
        out_shape=jax.ShapeDtypeStruct((G, 10), jnp.float32),
    )(t, s_part, s_part, batch2d, w1, b1, w2, b2)


def kernel(x, edge_index, batch, params):
    p = params
    ei = edge_index.astype(jnp.int32)
    pad_len = EC_PAD * CHUNK - E
    pad_iota = jnp.arange(pad_len, dtype=jnp.int32)
    srcs = jnp.concatenate([ei[0], pad_iota % N]) * 2
    dsts = jnp.concatenate([ei[1], N + pad_iota % (N_PAD - N)])
    epad = jnp.stack([srcs, dsts]).reshape(2, NW, SLAB, CHUNK)
    zeros = jnp.zeros((RPT, F_OUT), jnp.float32)
    batch2d = batch.astype(jnp.int32).reshape(1, N)

    r = lambda b: b.reshape(1, F_OUT)
    h_args = (x,)
    fuse = False
    for i in ("1", "2", "3"):
        wc_pad = jnp.pad(p["Wc" + i], ((0, 0), (0, 128 - F_OUT)))
        u = _mm_u(h_args, wc_pad, fuse).reshape(2 * N, F_OUT)
        s_part = _sc_scatter()(u, epad, zeros)
        t = _mm_t(h_args,
                  p["W" + i + "1"], p["W" + i + "2"], p["W" + i + "3"],
                  r(p["b" + i + "1"]), r(p["b" + i + "2"]),
                  r(p["b" + i + "3"]), r(p["bc" + i]), fuse)
        h_args = (t, s_part, s_part)
        fuse = True

    t, s_part, _ = h_args
    return _pool_fc(t, s_part, batch2d, p["Wfc1"], p["bfc1"].reshape(1, 32),
                    p["Wfc2"], p["bfc2"].reshape(1, 10))
